# baseline (speedup 1.0000x reference)
import functools

import jax
import jax.numpy as jnp
import numpy as np
from jax import lax
from jax.experimental import pallas as pl
from jax.experimental.pallas import tpu as pltpu
from jax.experimental.pallas import tpu_sc as plsc

N_VERTS = 100000
N_FACES = 200000

NC = 2
NS = 16
NW = NC * NS
L = 16

VW = 16
OW = 8
CHUNK = 128
FPW = N_FACES // NW
TAIL = FPW - 48 * CHUNK
F_PAD = N_FACES + 64

VROWS_PER_TILE = 6256
NV_PAD = NS * VROWS_PER_TILE


def _sc_body(vpad_hbm, fp_hbm, z_hbm, out_hbm,
             fci, idx0, idx1, idx2, idx3,
             rows00, rows01, rows02, rows10, rows11, rows12,
             nrm0, nrm1, acc,
             gsem0, gsem1, ssem0, ssem1):
  c = lax.axis_index("c")
  s = lax.axis_index("s")
  wid = c * NS + s

  idxb = (idx0, idx1, idx2, idx3)
  rows = ((rows00, rows01, rows02), (rows10, rows11, rows12))
  nrm = (nrm0, nrm1)
  gsem = (gsem0, gsem1)
  ssem = (ssem0, ssem1)

  iota = lax.broadcasted_iota(jnp.int32, (L,), 0)
  zf = jnp.zeros((L,), jnp.float32)
  col0 = jnp.zeros((L,), jnp.int32)
  col1 = col0 + 1
  col2 = col0 + 2

  for d in range(2):
    for j in range(CHUNK // L):
      for cc in range(3, VW):
        plsc.store_scatter(nrm[d], [j * L + iota, col0 + cc], zf)

  row0 = s * VROWS_PER_TILE
  pltpu.sync_copy(z_hbm.at[pl.ds(row0, VROWS_PER_TILE)],
                  acc.at[pl.ds(row0, VROWS_PER_TILE)])

  plsc.subcore_barrier()

  fbase = wid * FPW

  def stage(slot, foff):
    pltpu.sync_copy(fp_hbm.at[pl.ds(foff, CHUNK)], fci)
    for k in range(3):
      for j in range(CHUNK // L):
        r = j * L + iota
        t = plsc.load_gather(fci, [r, col0 + k])
        plsc.store_scatter(idxb[slot], [col0 + k, r], t)

  def fire_gathers(d, slot):
    for k in range(3):
      pltpu.async_copy(vpad_hbm.at[idxb[slot].at[k]], rows[d][k], gsem[d])

  def drain_gathers(d, slot):
    for k in range(3):
      pltpu.make_async_copy(vpad_hbm.at[idxb[slot].at[k]], rows[d][k],
                            gsem[d]).wait()

  def fire_scatters(d, slot):
    for k in range(3):
      pltpu.async_copy(nrm[d], acc.at[idxb[slot].at[k]], ssem[d], add=True)

  def drain_scatters(d, slot):
    for k in range(3):
      pltpu.make_async_copy(nrm[d], acc.at[idxb[slot].at[k]],
                            ssem[d]).wait()

  def compute(d):
    r0, r1, r2 = rows[d]
    for j in range(CHUNK // L):
      r = j * L + iota
      x0 = plsc.load_gather(r0, [r, col0])
      y0 = plsc.load_gather(r0, [r, col1])
      z0 = plsc.load_gather(r0, [r, col2])
      x1 = plsc.load_gather(r1, [r, col0])
      y1 = plsc.load_gather(r1, [r, col1])
      z1 = plsc.load_gather(r1, [r, col2])
      x2 = plsc.load_gather(r2, [r, col0])
      y2 = plsc.load_gather(r2, [r, col1])
      z2 = plsc.load_gather(r2, [r, col2])
      ux, uy, uz = x1 - x0, y1 - y0, z1 - z0
      vx, vy, vz = x2 - x0, y2 - y0, z2 - z0
      nx = (uy * vz - uz * vy) * 3.0
      ny = (uz * vx - ux * vz) * 3.0
      nz = (ux * vy - uy * vx) * 3.0
      plsc.store_scatter(nrm[d], [r, col0], nx)
      plsc.store_scatter(nrm[d], [r, col1], ny)
      plsc.store_scatter(nrm[d], [r, col2], nz)

  stage(0, fbase)
  fire_gathers(0, 0)
  stage(1, fbase + 1 * CHUNK)
  fire_gathers(1, 1)
  drain_gathers(0, 0)
  compute(0)
  fire_scatters(0, 0)
  stage(2, fbase + 2 * CHUNK)
  fire_gathers(0, 2)
  drain_gathers(1, 1)
  compute(1)
  fire_scatters(1, 1)
  stage(3, fbase + 3 * CHUNK)
  fire_gathers(1, 3)
  drain_scatters(0, 0)
  drain_gathers(0, 2)
  compute(0)
  fire_scatters(0, 2)
  stage(0, fbase + 4 * CHUNK)
  fire_gathers(0, 0)
  drain_scatters(1, 1)
  drain_gathers(1, 3)
  compute(1)
  fire_scatters(1, 3)

  def _super(k, _):
    cbase = fbase + 4 * k * CHUNK
    for j in range(4):
      d = j % 2
      stage((j + 1) % 4, cbase + (j + 1) * CHUNK)
      fire_gathers((j + 1) % 2, (j + 1) % 4)
      drain_scatters(d, (j + 2) % 4)
      drain_gathers(d, j)
      compute(d)
      fire_scatters(d, j)
    return 0

  lax.fori_loop(1, 12, _super, 0)

  drain_scatters(0, 2)
  drain_gathers(0, 0)
  compute(0)
  mtail = iota >= (TAIL - 6 * L)
  for cc in range(3):
    plsc.store_scatter(nrm[0], [6 * L + iota, col0 + cc], zf, mask=mtail)
    plsc.store_scatter(nrm[0], [7 * L + iota, col0 + cc], zf)
  fire_scatters(0, 0)
  drain_scatters(1, 3)
  drain_scatters(0, 0)

  plsc.subcore_barrier()

  pltpu.sync_copy(acc.at[pl.ds(row0, VROWS_PER_TILE), pl.ds(0, OW)],
                  out_hbm.at[c, pl.ds(row0, VROWS_PER_TILE)])


_sc_scatter = pl.kernel(
    _sc_body,
    out_type=jax.ShapeDtypeStruct((NC, NV_PAD, OW), jnp.float32),
    mesh=plsc.VectorSubcoreMesh(core_axis_name="c", subcore_axis_name="s"),
    compiler_params=pltpu.CompilerParams(
        needs_layout_passes=False, use_tc_tiling_on_sc=False),
    scratch_types=(
        [pltpu.VMEM((CHUNK, 3), jnp.int32)]
        + [pltpu.VMEM((3, CHUNK), jnp.int32)] * 4
        + [pltpu.VMEM((CHUNK, VW), jnp.float32)] * 6
        + [pltpu.VMEM((CHUNK, VW), jnp.float32)] * 2
        + [pltpu.VMEM_SHARED((NV_PAD, VW), jnp.float32)]
        + [pltpu.SemaphoreType.DMA] * 4
    ),
)


def _finish_body(p_ref, g_ref, o_ref):
  s = p_ref[0] + p_ref[1]
  t = s * s
  ss = jnp.dot(t, g_ref[...], preferred_element_type=jnp.float32)
  n = jnp.sqrt(ss)
  o_ref[...] = s / jnp.maximum(n, 1e-12)


_ROWS128 = NV_PAD * OW // 128

_finish = pl.pallas_call(
    _finish_body,
    out_shape=jax.ShapeDtypeStruct((_ROWS128, 128), jnp.float32),
)

_G = np.kron(np.eye(128 // OW, dtype=np.float32),
             np.ones((OW, OW), dtype=np.float32))


@jax.jit
def kernel(v, faces):
  vpad = jnp.pad(v, ((0, 0), (0, VW - 3)))
  fpad = jnp.pad(faces.astype(jnp.int32), ((0, F_PAD - N_FACES), (0, 0)))
  partials = _sc_scatter(vpad, fpad, jnp.zeros((NV_PAD, VW), jnp.float32))
  out = _finish(partials.reshape(NC, _ROWS128, 128), jnp.asarray(_G))
  return out.reshape(NV_PAD, OW)[:N_VERTS, :3]


# speedup vs baseline: 1.5658x; 1.5658x over previous
import functools

import jax
import jax.numpy as jnp
import numpy as np
from jax import lax
from jax.experimental import pallas as pl
from jax.experimental.pallas import tpu as pltpu
from jax.experimental.pallas import tpu_sc as plsc

N_VERTS = 100000
N_FACES = 200000

NC = 2
NS = 16
NW = NC * NS
L = 16

VW = 16
OW = 8
CHUNK = 128
CPW = 49
FPW = CPW * CHUNK
F_PAD = NW * FPW

VROWS_PER_TILE = 6256
NV_PAD = NS * VROWS_PER_TILE


def _sc_body(vpad_hbm, fc_hbm, z_hbm, out_hbm,
             idx0, idx1, idx2, idx3,
             rows00, rows01, rows02, rows10, rows11, rows12,
             nrm0, nrm1, acc,
             gsem0, gsem1, ssem0, ssem1):
  c = lax.axis_index("c")
  s = lax.axis_index("s")
  wid = c * NS + s

  idxb = (idx0, idx1, idx2, idx3)
  rows = ((rows00, rows01, rows02), (rows10, rows11, rows12))
  nrm = (nrm0, nrm1)
  gsem = (gsem0, gsem1)
  ssem = (ssem0, ssem1)

  iota = lax.broadcasted_iota(jnp.int32, (L,), 0)
  zf = jnp.zeros((L,), jnp.float32)
  col0 = jnp.zeros((L,), jnp.int32)
  col1 = col0 + 1
  col2 = col0 + 2

  for d in range(2):
    for j in range(CHUNK // L):
      for cc in range(3, VW):
        plsc.store_scatter(nrm[d], [j * L + iota, col0 + cc], zf)

  row0 = s * VROWS_PER_TILE
  pltpu.sync_copy(z_hbm.at[pl.ds(row0, VROWS_PER_TILE)],
                  acc.at[pl.ds(row0, VROWS_PER_TILE)])

  plsc.subcore_barrier()

  cid0 = wid * CPW

  def stage(slot, cid):
    pltpu.sync_copy(fc_hbm.at[cid], idxb[slot])

  def fire_gathers(d, slot):
    for k in range(3):
      pltpu.async_copy(vpad_hbm.at[idxb[slot].at[k]], rows[d][k], gsem[d])

  def drain_gathers(d, slot):
    for k in range(3):
      pltpu.make_async_copy(vpad_hbm.at[idxb[slot].at[k]], rows[d][k],
                            gsem[d]).wait()

  def fire_scatters(d, slot):
    for k in range(3):
      pltpu.async_copy(nrm[d], acc.at[idxb[slot].at[k]], ssem[d], add=True)

  def drain_scatters(d, slot):
    for k in range(3):
      pltpu.make_async_copy(nrm[d], acc.at[idxb[slot].at[k]],
                            ssem[d]).wait()

  def compute(d):
    r0, r1, r2 = rows[d]
    for j in range(CHUNK // L):
      r = j * L + iota
      x0 = plsc.load_gather(r0, [r, col0])
      y0 = plsc.load_gather(r0, [r, col1])
      z0 = plsc.load_gather(r0, [r, col2])
      x1 = plsc.load_gather(r1, [r, col0])
      y1 = plsc.load_gather(r1, [r, col1])
      z1 = plsc.load_gather(r1, [r, col2])
      x2 = plsc.load_gather(r2, [r, col0])
      y2 = plsc.load_gather(r2, [r, col1])
      z2 = plsc.load_gather(r2, [r, col2])
      ux, uy, uz = x1 - x0, y1 - y0, z1 - z0
      vx, vy, vz = x2 - x0, y2 - y0, z2 - z0
      nx = (uy * vz - uz * vy) * 3.0
      ny = (uz * vx - ux * vz) * 3.0
      nz = (ux * vy - uy * vx) * 3.0
      plsc.store_scatter(nrm[d], [r, col0], nx)
      plsc.store_scatter(nrm[d], [r, col1], ny)
      plsc.store_scatter(nrm[d], [r, col2], nz)

  stage(0, cid0)
  fire_gathers(0, 0)
  stage(1, cid0 + 1)
  fire_gathers(1, 1)
  drain_gathers(0, 0)
  compute(0)
  fire_scatters(0, 0)
  stage(2, cid0 + 2)
  fire_gathers(0, 2)
  drain_gathers(1, 1)
  compute(1)
  fire_scatters(1, 1)
  stage(3, cid0 + 3)
  fire_gathers(1, 3)
  drain_scatters(0, 0)
  drain_gathers(0, 2)
  compute(0)
  fire_scatters(0, 2)
  stage(0, cid0 + 4)
  fire_gathers(0, 0)
  drain_scatters(1, 1)
  drain_gathers(1, 3)
  compute(1)
  fire_scatters(1, 3)

  def _super(k, _):
    cbase = cid0 + 4 * k
    for j in range(4):
      d = j % 2
      stage((j + 1) % 4, cbase + j + 1)
      fire_gathers((j + 1) % 2, (j + 1) % 4)
      drain_scatters(d, (j + 2) % 4)
      drain_gathers(d, j)
      compute(d)
      fire_scatters(d, j)
    return 0

  lax.fori_loop(1, 12, _super, 0)

  drain_scatters(0, 2)
  drain_gathers(0, 0)
  compute(0)
  fire_scatters(0, 0)
  drain_scatters(1, 3)
  drain_scatters(0, 0)

  plsc.subcore_barrier()

  pltpu.sync_copy(acc.at[pl.ds(row0, VROWS_PER_TILE), pl.ds(0, OW)],
                  out_hbm.at[c, pl.ds(row0, VROWS_PER_TILE)])


_sc_scatter = pl.kernel(
    _sc_body,
    out_type=jax.ShapeDtypeStruct((NC, NV_PAD, OW), jnp.float32),
    mesh=plsc.VectorSubcoreMesh(core_axis_name="c", subcore_axis_name="s"),
    compiler_params=pltpu.CompilerParams(
        needs_layout_passes=False, use_tc_tiling_on_sc=False),
    scratch_types=(
        [pltpu.VMEM((3, CHUNK), jnp.int32)] * 4
        + [pltpu.VMEM((CHUNK, VW), jnp.float32)] * 6
        + [pltpu.VMEM((CHUNK, VW), jnp.float32)] * 2
        + [pltpu.VMEM_SHARED((NV_PAD, VW), jnp.float32)]
        + [pltpu.SemaphoreType.DMA] * 4
    ),
)


def _finish_body(p_ref, g_ref, o_ref):
  s = p_ref[0] + p_ref[1]
  t = s * s
  ss = jnp.dot(t, g_ref[...], preferred_element_type=jnp.float32)
  n = jnp.sqrt(ss)
  o_ref[...] = s / jnp.maximum(n, 1e-12)


_ROWS128 = NV_PAD * OW // 128

_finish = pl.pallas_call(
    _finish_body,
    out_shape=jax.ShapeDtypeStruct((_ROWS128, 128), jnp.float32),
)

_G = np.kron(np.eye(128 // OW, dtype=np.float32),
             np.ones((OW, OW), dtype=np.float32))


@jax.jit
def kernel(v, faces):
  vpad = jnp.pad(v, ((0, 0), (0, VW - 3)))
  fpad = jnp.pad(faces.astype(jnp.int32), ((0, F_PAD - N_FACES), (0, 0)))
  fc = fpad.reshape(NW * CPW, CHUNK, 3).transpose(0, 2, 1)
  partials = _sc_scatter(vpad, fc, jnp.zeros((NV_PAD, VW), jnp.float32))
  out = _finish(partials.reshape(NC, _ROWS128, 128), jnp.asarray(_G))
  return out.reshape(NV_PAD, OW)[:N_VERTS, :3]


# speedup vs baseline: 1.6239x; 1.0371x over previous
import functools

import jax
import jax.numpy as jnp
import numpy as np
from jax import lax
from jax.experimental import pallas as pl
from jax.experimental.pallas import tpu as pltpu
from jax.experimental.pallas import tpu_sc as plsc

N_VERTS = 100000
N_FACES = 200000

NC = 2
NS = 16
NW = NC * NS
L = 16

VW = 16
OW = 8
CHUNK = 128
CPW = 49
FPW = CPW * CHUNK
F_PAD = NW * FPW

VROWS_PER_TILE = 6256
NV_PAD = NS * VROWS_PER_TILE


def _sc_body(vpad_hbm, fc_hbm, z_hbm, out_hbm,
             idx0, idx1, idx2, idx3,
             rows00, rows01, rows02, rows10, rows11, rows12,
             nrm0, nrm1, acc,
             gsem0, gsem1, ssem0, ssem1, isem0, isem1):
  c = lax.axis_index("c")
  s = lax.axis_index("s")
  wid = c * NS + s

  idxb = (idx0, idx1, idx2, idx3)
  rows = ((rows00, rows01, rows02), (rows10, rows11, rows12))
  nrm = (nrm0, nrm1)
  gsem = (gsem0, gsem1)
  ssem = (ssem0, ssem1)
  isem = (isem0, isem1)

  iota = lax.broadcasted_iota(jnp.int32, (L,), 0)
  zf = jnp.zeros((L,), jnp.float32)
  col0 = jnp.zeros((L,), jnp.int32)
  col1 = col0 + 1
  col2 = col0 + 2

  for d in range(2):
    for j in range(CHUNK // L):
      for cc in range(3, VW):
        plsc.store_scatter(nrm[d], [j * L + iota, col0 + cc], zf)

  row0 = s * VROWS_PER_TILE
  pltpu.sync_copy(z_hbm.at[pl.ds(row0, VROWS_PER_TILE)],
                  acc.at[pl.ds(row0, VROWS_PER_TILE)])

  plsc.subcore_barrier()

  cid0 = wid * CPW

  def stage_async(slot, cid, m):
    pltpu.async_copy(fc_hbm.at[cid], idxb[slot], isem[m])

  def stage_drain(slot, cid, m):
    pltpu.make_async_copy(fc_hbm.at[cid], idxb[slot], isem[m]).wait()

  def fire_gathers(d, slot):
    for k in range(3):
      pltpu.async_copy(vpad_hbm.at[idxb[slot].at[k]], rows[d][k], gsem[d])

  def drain_gathers(d, slot):
    for k in range(3):
      pltpu.make_async_copy(vpad_hbm.at[idxb[slot].at[k]], rows[d][k],
                            gsem[d]).wait()

  def fire_scatters(d, slot):
    for k in range(3):
      pltpu.async_copy(nrm[d], acc.at[idxb[slot].at[k]], ssem[d], add=True)

  def drain_scatters(d, slot):
    for k in range(3):
      pltpu.make_async_copy(nrm[d], acc.at[idxb[slot].at[k]],
                            ssem[d]).wait()

  def compute(d):
    r0, r1, r2 = rows[d]
    for j in range(CHUNK // L):
      r = j * L + iota
      x0 = plsc.load_gather(r0, [r, col0])
      y0 = plsc.load_gather(r0, [r, col1])
      z0 = plsc.load_gather(r0, [r, col2])
      x1 = plsc.load_gather(r1, [r, col0])
      y1 = plsc.load_gather(r1, [r, col1])
      z1 = plsc.load_gather(r1, [r, col2])
      x2 = plsc.load_gather(r2, [r, col0])
      y2 = plsc.load_gather(r2, [r, col1])
      z2 = plsc.load_gather(r2, [r, col2])
      ux, uy, uz = x1 - x0, y1 - y0, z1 - z0
      vx, vy, vz = x2 - x0, y2 - y0, z2 - z0
      nx = (uy * vz - uz * vy) * 3.0
      ny = (uz * vx - ux * vz) * 3.0
      nz = (ux * vy - uy * vx) * 3.0
      plsc.store_scatter(nrm[d], [r, col0], nx)
      plsc.store_scatter(nrm[d], [r, col1], ny)
      plsc.store_scatter(nrm[d], [r, col2], nz)

  stage_async(0, cid0, 0)
  stage_async(1, cid0 + 1, 1)
  stage_drain(0, cid0, 0)
  fire_gathers(0, 0)
  stage_async(2, cid0 + 2, 0)
  stage_drain(1, cid0 + 1, 1)
  fire_gathers(1, 1)
  drain_gathers(0, 0)
  compute(0)
  fire_scatters(0, 0)
  stage_async(3, cid0 + 3, 1)
  stage_drain(2, cid0 + 2, 0)
  fire_gathers(0, 2)
  drain_gathers(1, 1)
  compute(1)
  fire_scatters(1, 1)
  drain_scatters(0, 0)
  stage_async(0, cid0 + 4, 0)
  stage_drain(3, cid0 + 3, 1)
  fire_gathers(1, 3)
  drain_gathers(0, 2)
  compute(0)
  fire_scatters(0, 2)
  drain_scatters(1, 1)
  stage_async(1, cid0 + 5, 1)
  stage_drain(0, cid0 + 4, 0)
  fire_gathers(0, 0)
  drain_gathers(1, 3)
  compute(1)
  fire_scatters(1, 3)

  def _super(k, _):
    cbase = cid0 + 4 * k
    for j in range(4):
      d = j % 2
      drain_scatters(d, (j + 2) % 4)
      if j == 3:
        @pl.when(k < 11)
        def _():
          stage_async((j + 2) % 4, cbase + j + 2, d)
      else:
        stage_async((j + 2) % 4, cbase + j + 2, d)
      stage_drain((j + 1) % 4, cbase + j + 1, (j + 1) % 2)
      fire_gathers((j + 1) % 2, (j + 1) % 4)
      drain_gathers(d, j)
      compute(d)
      fire_scatters(d, j)
    return 0

  lax.fori_loop(1, 12, _super, 0)

  drain_scatters(0, 2)
  drain_gathers(0, 0)
  compute(0)
  fire_scatters(0, 0)
  drain_scatters(1, 3)
  drain_scatters(0, 0)

  plsc.subcore_barrier()

  pltpu.sync_copy(acc.at[pl.ds(row0, VROWS_PER_TILE), pl.ds(0, OW)],
                  out_hbm.at[c, pl.ds(row0, VROWS_PER_TILE)])


_sc_scatter = pl.kernel(
    _sc_body,
    out_type=jax.ShapeDtypeStruct((NC, NV_PAD, OW), jnp.float32),
    mesh=plsc.VectorSubcoreMesh(core_axis_name="c", subcore_axis_name="s"),
    compiler_params=pltpu.CompilerParams(
        needs_layout_passes=False, use_tc_tiling_on_sc=False),
    scratch_types=(
        [pltpu.VMEM((3, CHUNK), jnp.int32)] * 4
        + [pltpu.VMEM((CHUNK, VW), jnp.float32)] * 6
        + [pltpu.VMEM((CHUNK, VW), jnp.float32)] * 2
        + [pltpu.VMEM_SHARED((NV_PAD, VW), jnp.float32)]
        + [pltpu.SemaphoreType.DMA] * 6
    ),
)


def _finish_body(p_ref, g_ref, o_ref):
  s = p_ref[0] + p_ref[1]
  t = s * s
  ss = jnp.dot(t, g_ref[...], preferred_element_type=jnp.float32)
  n = jnp.sqrt(ss)
  o_ref[...] = s / jnp.maximum(n, 1e-12)


_ROWS128 = NV_PAD * OW // 128

_finish = pl.pallas_call(
    _finish_body,
    out_shape=jax.ShapeDtypeStruct((_ROWS128, 128), jnp.float32),
)

_G = np.kron(np.eye(128 // OW, dtype=np.float32),
             np.ones((OW, OW), dtype=np.float32))


@jax.jit
def kernel(v, faces):
  vpad = jnp.pad(v, ((0, 0), (0, VW - 3)))
  fpad = jnp.pad(faces.astype(jnp.int32), ((0, F_PAD - N_FACES), (0, 0)))
  fc = fpad.reshape(NW * CPW, CHUNK, 3).transpose(0, 2, 1)
  partials = _sc_scatter(vpad, fc, jnp.zeros((NV_PAD, VW), jnp.float32))
  out = _finish(partials.reshape(NC, _ROWS128, 128), jnp.asarray(_G))
  return out.reshape(NV_PAD, OW)[:N_VERTS, :3]
